# baseline (speedup 1.0000x reference)
import functools

import jax
import jax.numpy as jnp
from jax import lax
from jax.experimental import pallas as pl
from jax.experimental.pallas import tpu as pltpu
from jax.experimental.pallas import tpu_sc as plsc

NUM_UNITS = 32
SCALE = float(NUM_UNITS) ** 0.5

_NC = 2
_NS = 16
_NW = _NC * _NS

_mesh = plsc.VectorSubcoreMesh(core_axis_name="c", subcore_axis_name="s")


@functools.lru_cache(maxsize=None)
def _build_retile(vocab: int, units: int):
    nfull = vocab // 128
    tail = vocab - nfull * 128
    vpad = (vocab + 127) // 128 * 128
    kmax = (nfull + _NW - 1) // _NW

    @functools.partial(
        pl.kernel,
        out_type=jax.ShapeDtypeStruct((vpad * units // 128, 128), jnp.float32),
        mesh=_mesh,
        scratch_types=[
            pltpu.VMEM((2, units, 128), jnp.float32),
            pltpu.VMEM((2, units, 128), jnp.float32),
            pltpu.VMEM((max(1, tail * units // 128), 128), jnp.float32),
            [pltpu.SemaphoreType.DMA] * 2,
            [pltpu.SemaphoreType.DMA] * 2,
        ],
        compiler_params=pltpu.CompilerParams(
            use_tc_tiling_on_sc=True, needs_layout_passes=False
        ),
    )
    def retile(tblt_hbm, tail_hbm, out_hbm, bin_v, bout_v, tin_v, gsems, osems):
        wid = lax.axis_index("s") * _NC + lax.axis_index("c")
        iota = lax.iota(jnp.int32, 16)

        def in_copy(k, b):
            blk = wid + k * _NW
            return pltpu.make_async_copy(
                tblt_hbm.at[:, pl.ds(pl.multiple_of(blk * 128, 128), 128)],
                bin_v.at[b],
                gsems[b],
            )

        def out_copy(k, b):
            blk = wid + k * _NW
            return pltpu.make_async_copy(
                bout_v.at[b],
                out_hbm.at[pl.ds(pl.multiple_of(blk * units, 8), units)],
                osems[b],
            )

        def transpose_block(src, dst, ncol):
            for c in range(ncol):
                cvec = jnp.full((16,), c, jnp.int32)
                for u0 in range(0, units, 16):
                    v = plsc.load_gather(src, [iota + u0, cvec]) * SCALE
                    dst[(c * units + u0) // 128,
                        pl.ds((c * units + u0) % 128, 16)] = v

        in_copy(0, 0).start()

        def step(k, b):
            b2 = 1 - b
            blk = wid + k * _NW

            @pl.when(blk + _NW < nfull)
            def _prefetch():
                @pl.when(k >= 1)
                def _wait_write():
                    out_copy(k - 1, b2).wait()

                in_copy(k + 1, b2).start()

            @pl.when(blk < nfull)
            def _process():
                in_copy(k, b).wait()
                transpose_block(bin_v.at[b], bout_v.at[b], 128)
                out_copy(k, b).start()

        def body(k2, carry):
            for b in range(2):
                step(k2 * 2 + b, b)
            return carry

        lax.fori_loop(0, kmax // 2, body, 0)
        if kmax % 2:
            step(kmax - 1, (kmax - 1) % 2)

        nk = lax.div(nfull - 1 - wid, _NW) + 1
        for b in range(2):
            out_copy(lax.div(nk - 1 - b, 2) * 2 + b, b).wait()

        if tail:
            @pl.when(wid == 0)
            def _tail():
                pltpu.sync_copy(tail_hbm, tin_v)
                pltpu.sync_copy(
                    tin_v,
                    out_hbm.at[pl.ds(nfull * units, units * tail // 128)],
                )

    return retile


@functools.lru_cache(maxsize=None)
def _build_gather(b0: int, b1: int, vpad: int, units: int):
    bpw = (b0 // _NW) * b1
    rpw = b0 // _NW
    nbuf = 4
    nt2 = b1 // nbuf
    nub = units // 8

    @functools.partial(
        pl.kernel,
        out_type=jax.ShapeDtypeStruct((b1, nub, b0 // 128, 8, 128), jnp.float32),
        mesh=_mesh,
        scratch_types=[
            pltpu.VMEM((bpw,), jnp.int32),
            pltpu.VMEM((nbuf, 128), jnp.int32),
            pltpu.VMEM((nbuf, 128, units), jnp.float32),
            pltpu.VMEM((nbuf, nub, 8, 128), jnp.float32),
            [pltpu.SemaphoreType.DMA] * nbuf,
            [pltpu.SemaphoreType.DMA] * nbuf,
        ],
        compiler_params=pltpu.CompilerParams(
            use_tc_tiling_on_sc=False, needs_layout_passes=False
        ),
    )
    def gather(tbl_hbm, idx_hbm, out_hbm, idx_all, idxt_v, rows_v, bt_v,
               gsems, osems):
        wid = lax.axis_index("s") * _NC + lax.axis_index("c")
        iota = lax.iota(jnp.int32, 16)
        iota_b1 = iota * b1

        pltpu.sync_copy(
            idx_hbm.at[pl.ds(pl.multiple_of(wid * bpw, 8), bpw)], idx_all
        )

        def build_and_fire(t, b):
            for s in range(rpw // 16):
                p = iota_b1 + (s * 16 * b1 + t)
                idxt_v[b, pl.ds(s * 16, 16)] = plsc.load_gather(idx_all, [p])
            pltpu.make_async_copy(
                tbl_hbm.at[idxt_v.at[b]], rows_v.at[b], gsems[b]
            ).start()

        def write_copies(t, b):
            return [
                pltpu.make_async_copy(
                    bt_v.at[b, ub], out_hbm.at[t, ub, wid], osems[b]
                )
                for ub in range(nub)
            ]

        def transpose_chunk(b):
            for s in range(rpw // 16):
                rvec = iota + (s * 16)
                for u in range(units):
                    v = plsc.load_gather(
                        rows_v.at[b], [rvec, jnp.full((16,), u, jnp.int32)]
                    )
                    bt_v[b, u // 8, u % 8, pl.ds(s * 16, 16)] = v

        build_and_fire(0, 0)
        build_and_fire(1, 1)

        def body(t2, carry):
            for b in range(nbuf):
                t = t2 * nbuf + b
                b2 = (b + 2) % nbuf
                if b < 2:
                    @pl.when(t2 > 0)
                    def _w():
                        for c in write_copies(t - 2, b2):
                            c.wait()

                    build_and_fire(t + 2, b2)
                else:
                    @pl.when(t + 2 < b1)
                    def _wf():
                        for c in write_copies(t - 2, b2):
                            c.wait()
                        build_and_fire(t + 2, b2)

                pltpu.make_async_copy(
                    tbl_hbm.at[idxt_v.at[b]], rows_v.at[b], gsems[b]
                ).wait()
                transpose_chunk(b)
                for c in write_copies(t, b):
                    c.start()
            return carry

        lax.fori_loop(0, nt2, body, 0)

        for b in range(nbuf):
            for c in write_copies(b1 - nbuf + b, b):
                c.wait()

    return gather


def kernel(inputs, lookup_table):
    b0, b1 = inputs.shape
    vocab, units = lookup_table.shape
    vpad = (vocab + 127) // 128 * 128
    idx = inputs.reshape(b0 * b1).astype(jnp.int32)
    tblt = jnp.transpose(lookup_table)
    nfull = vocab // 128
    tail = vocab - nfull * 128
    if tail:
        tail2d = (lookup_table[nfull * 128:] * SCALE).reshape(
            tail * units // 128, 128
        )
    else:
        tail2d = jnp.zeros((1, 128), jnp.float32)
    rm = _build_retile(vocab, units)(tblt, tail2d)
    tbl_lin = rm.reshape(vpad, units)
    out5d = _build_gather(b0, b1, vpad, units)(tbl_lin, idx)
    return jnp.transpose(out5d, (2, 4, 0, 1, 3)).reshape(b0, b1, units)


# speedup vs baseline: 2.7867x; 2.7867x over previous
import functools

import jax
import jax.numpy as jnp
import numpy as np
from jax import lax
from jax.experimental import pallas as pl
from jax.experimental.pallas import tpu as pltpu
from jax.experimental.pallas import tpu_sc as plsc

NUM_UNITS = 32
SCALE = float(NUM_UNITS) ** 0.5

_NC = 2
_NS = 16
_NW = _NC * _NS

_mesh = plsc.VectorSubcoreMesh(core_axis_name="c", subcore_axis_name="s")


@functools.lru_cache(maxsize=None)
def _build_retile(vocab: int, units: int):
    nfull = vocab // 128
    tail = vocab - nfull * 128
    vpad = (vocab + 127) // 128 * 128
    kmax = (nfull + _NW - 1) // _NW

    @functools.partial(
        pl.kernel,
        out_type=jax.ShapeDtypeStruct((vpad * units // 128, 128), jnp.float32),
        mesh=_mesh,
        scratch_types=[
            pltpu.VMEM((2, units, 128), jnp.float32),
            pltpu.VMEM((2, units, 128), jnp.float32),
            pltpu.VMEM((max(1, tail * units // 128), 128), jnp.float32),
            [pltpu.SemaphoreType.DMA] * 2,
            [pltpu.SemaphoreType.DMA] * 2,
        ],
        compiler_params=pltpu.CompilerParams(
            use_tc_tiling_on_sc=True, needs_layout_passes=False
        ),
    )
    def retile(tblt_hbm, tail_hbm, out_hbm, bin_v, bout_v, tin_v, gsems, osems):
        wid = lax.axis_index("s") * _NC + lax.axis_index("c")
        iota = lax.iota(jnp.int32, 16)

        def in_copy(k, b):
            blk = wid + k * _NW
            return pltpu.make_async_copy(
                tblt_hbm.at[:, pl.ds(pl.multiple_of(blk * 128, 128), 128)],
                bin_v.at[b],
                gsems[b],
            )

        def out_copy(k, b):
            blk = wid + k * _NW
            return pltpu.make_async_copy(
                bout_v.at[b],
                out_hbm.at[pl.ds(pl.multiple_of(blk * units, 8), units)],
                osems[b],
            )

        perms = [lax.rem(iota + k, 16) for k in range(16)]
        fvecs = [perms[k] * units + iota for k in range(16)]

        def transpose_block(src, dst, ncol):
            def tile(c16, carry):
                c0 = c16 * 16
                for u0 in range(0, units, 16):
                    for k in range(16):
                        v = plsc.load_gather(
                            src, [iota + u0, perms[k] + c0]
                        ) * SCALE
                        f = fvecs[k] + (c0 * units + u0)
                        plsc.store_scatter(
                            dst, [lax.div(f, 128), lax.rem(f, 128)], v
                        )
                return carry

            lax.fori_loop(0, ncol // 16, tile, 0)

        in_copy(0, 0).start()

        def step(k, b):
            b2 = 1 - b
            blk = wid + k * _NW

            @pl.when(blk + _NW < nfull)
            def _prefetch():
                @pl.when(k >= 1)
                def _wait_write():
                    out_copy(k - 1, b2).wait()

                in_copy(k + 1, b2).start()

            @pl.when(blk < nfull)
            def _process():
                in_copy(k, b).wait()
                transpose_block(bin_v.at[b], bout_v.at[b], 128)
                out_copy(k, b).start()

        def body(k2, carry):
            for b in range(2):
                step(k2 * 2 + b, b)
            return carry

        lax.fori_loop(0, kmax // 2, body, 0)
        if kmax % 2:
            step(kmax - 1, (kmax - 1) % 2)

        nk = lax.div(nfull - 1 - wid, _NW) + 1
        for b in range(2):
            out_copy(lax.div(nk - 1 - b, 2) * 2 + b, b).wait()

        if tail:
            @pl.when(wid == 0)
            def _tail():
                pltpu.sync_copy(tail_hbm, tin_v)
                pltpu.sync_copy(
                    tin_v,
                    out_hbm.at[pl.ds(nfull * units, units * tail // 128)],
                )

    return retile


@functools.lru_cache(maxsize=None)
def _build_gather(b0: int, b1: int, vpad: int, units: int):
    bpw = (b0 // _NW) * b1
    rpw = b0 // _NW
    nbuf = 4
    nt2 = b1 // nbuf
    nub = units // 8

    @functools.partial(
        pl.kernel,
        out_type=jax.ShapeDtypeStruct((b1, nub, b0 // 128, 8, 128), jnp.float32),
        mesh=_mesh,
        scratch_types=[
            pltpu.VMEM((bpw,), jnp.int32),
            pltpu.VMEM((nbuf, 128), jnp.int32),
            pltpu.VMEM((nbuf, 128, units), jnp.float32),
            pltpu.VMEM((nbuf, nub, 8, 128), jnp.float32),
            [pltpu.SemaphoreType.DMA] * nbuf,
            [pltpu.SemaphoreType.DMA] * nbuf,
        ],
        compiler_params=pltpu.CompilerParams(
            use_tc_tiling_on_sc=False, needs_layout_passes=False
        ),
    )
    def gather(tbl_hbm, idx_hbm, out_hbm, idx_all, idxt_v, rows_v, bt_v,
               gsems, osems):
        wid = lax.axis_index("s") * _NC + lax.axis_index("c")
        iota = lax.iota(jnp.int32, 16)
        iota_b1 = iota * b1

        pltpu.sync_copy(
            idx_hbm.at[pl.ds(pl.multiple_of(wid * bpw, 8), bpw)], idx_all
        )

        def build_and_fire(t, b):
            for s in range(rpw // 16):
                p = iota_b1 + (s * 16 * b1 + t)
                idxt_v[b, pl.ds(s * 16, 16)] = plsc.load_gather(idx_all, [p])
            pltpu.make_async_copy(
                tbl_hbm.at[idxt_v.at[b]], rows_v.at[b], gsems[b]
            ).start()

        def write_copies(t, b):
            return [
                pltpu.make_async_copy(
                    bt_v.at[b, ub], out_hbm.at[t, ub, wid], osems[b]
                )
                for ub in range(nub)
            ]

        perms = [lax.rem(iota + k, 16) for k in range(16)]
        d0s = {
            (u0, k): lax.div(perms[k] + u0, 8)
            for u0 in range(0, units, 16) for k in range(16)
        }
        d1s = {
            (u0, k): lax.rem(perms[k] + u0, 8)
            for u0 in range(0, units, 16) for k in range(16)
        }

        def transpose_chunk(b):
            def tile(s, carry):
                rvec = iota + s * 16
                for u0 in range(0, units, 16):
                    for k in range(16):
                        v = plsc.load_gather(
                            rows_v.at[b], [rvec, perms[k] + u0]
                        )
                        plsc.store_scatter(
                            bt_v.at[b], [d0s[(u0, k)], d1s[(u0, k)], rvec], v
                        )
                return carry

            lax.fori_loop(0, rpw // 16, tile, 0)

        build_and_fire(0, 0)
        build_and_fire(1, 1)

        def body(t2, carry):
            for b in range(nbuf):
                t = t2 * nbuf + b
                b2 = (b + 2) % nbuf
                if b < 2:
                    @pl.when(t2 > 0)
                    def _w():
                        for c in write_copies(t - 2, b2):
                            c.wait()

                    build_and_fire(t + 2, b2)
                else:
                    @pl.when(t + 2 < b1)
                    def _wf():
                        for c in write_copies(t - 2, b2):
                            c.wait()
                        build_and_fire(t + 2, b2)

                pltpu.make_async_copy(
                    tbl_hbm.at[idxt_v.at[b]], rows_v.at[b], gsems[b]
                ).wait()
                transpose_chunk(b)
                for c in write_copies(t, b):
                    c.start()
            return carry

        lax.fori_loop(0, nt2, body, 0)

        for b in range(nbuf):
            for c in write_copies(b1 - nbuf + b, b):
                c.wait()

    return gather


def kernel(inputs, lookup_table):
    b0, b1 = inputs.shape
    vocab, units = lookup_table.shape
    vpad = (vocab + 127) // 128 * 128
    idx = inputs.reshape(b0 * b1).astype(jnp.int32)
    tblt = jnp.transpose(lookup_table)
    nfull = vocab // 128
    tail = vocab - nfull * 128
    if tail:
        tail2d = (lookup_table[nfull * 128:] * SCALE).reshape(
            tail * units // 128, 128
        )
    else:
        tail2d = jnp.zeros((1, 128), jnp.float32)
    rm = _build_retile(vocab, units)(tblt, tail2d)
    tbl_lin = rm.reshape(vpad, units)
    out5d = _build_gather(b0, b1, vpad, units)(tbl_lin, idx)
    return jnp.transpose(out5d, (2, 4, 0, 1, 3)).reshape(b0, b1, units)
